# baseline (speedup 1.0000x reference)
import functools
import math

import jax
import jax.numpy as jnp
from jax import lax
from jax.experimental import pallas as pl
from jax.experimental.pallas import tpu as pltpu
import jax.experimental.pallas.tpu_sc as plsc

EPS = 1e-5
_INV_SQRT = 1.0 / math.sqrt(1.0 + EPS)


def _sift_select_body(keys_ref, q_ref, out_ref, *, radius, N, TQ):
    k3n = keys_ref[0]
    q3 = q_ref[0]
    dot = jax.lax.dot(q3, k3n, preferred_element_type=jnp.float32)
    sq = jnp.sum(q3 * q3, axis=1, keepdims=True)
    sk = jnp.sum(k3n * k3n, axis=0, keepdims=True)
    d2 = sq + sk - 2.0 * dot

    gx = (k3n[0:1, :] > q3[:, 0:1]).astype(jnp.int32)
    gy = (k3n[1:2, :] > q3[:, 1:2]).astype(jnp.int32)
    gz = (k3n[2:3, :] > q3[:, 2:3]).astype(jnp.int32)
    octant = gx * 4 + gy * 2 + gz

    qbase = pl.program_id(1) * TQ
    col = jax.lax.broadcasted_iota(jnp.int32, (TQ, N), 1)
    rowg = jax.lax.broadcasted_iota(jnp.int32, (TQ, N), 0) + qbase
    valid = (d2 <= radius * radius) & (col != rowg)
    self_idx = (jax.lax.broadcasted_iota(jnp.int32, (TQ, 1), 0) + qbase)[:, 0]

    colf = col.astype(jnp.float32)
    INF = jnp.float32(jnp.inf)
    for o in range(8):
        vo = valid & (octant == o)
        d_o = jnp.where(vo, d2, INF)
        m_o = jnp.min(d_o, axis=1, keepdims=True)
        j_o = jnp.min(jnp.where(vo & (d2 == m_o), colf, jnp.float32(N)),
                      axis=1)
        found = m_o[:, 0] < INF
        out_ref[0, o, :] = jnp.where(found, j_o.astype(jnp.int32), self_idx)


def _sift_select(xyzT, xyz, radius):
    B, _, N = xyzT.shape
    TQ = min(N, 256)
    grid = (B, N // TQ)
    return pl.pallas_call(
        functools.partial(_sift_select_body, radius=radius, N=N, TQ=TQ),
        grid=grid,
        in_specs=[
            pl.BlockSpec((1, 3, N), lambda b, t: (b, 0, 0)),
            pl.BlockSpec((1, TQ, 3), lambda b, t: (b, t, 0)),
        ],
        out_specs=pl.BlockSpec((1, 8, TQ), lambda b, t: (b, 0, t)),
        out_shape=jax.ShapeDtypeStruct((B, 8, N), jnp.int32),
    )(xyzT, xyz)


def _fps_body(xyzg_ref, out_ref, *, npoint, N):
    x = xyzg_ref[0, 0]
    y = xyzg_ref[0, 1]
    z = xyzg_ref[0, 2]
    N8 = N // 8
    giota = (jax.lax.broadcasted_iota(jnp.int32, (8, N8), 0) * N8
             + jax.lax.broadcasted_iota(jnp.int32, (8, N8), 1))
    np_iota = jax.lax.broadcasted_iota(jnp.int32, (1, npoint), 1)

    def body(i, state):
        cent, dist, far = state
        cent = jnp.where(np_iota == i, far, cent)
        m = giota == far
        cx = jnp.sum(jnp.where(m, x, 0.0))
        cy = jnp.sum(jnp.where(m, y, 0.0))
        cz = jnp.sum(jnp.where(m, z, 0.0))
        dx = x - cx
        dy = y - cy
        dz = z - cz
        dn = dx * dx + dy * dy + dz * dz
        dist = jnp.minimum(dist, dn)
        mx = jnp.max(dist)
        far_new = jnp.min(jnp.where(dist == mx, giota, jnp.int32(N)))
        return cent, dist, far_new

    init = (jnp.zeros((1, npoint), jnp.int32),
            jnp.full((8, N8), 1e10, jnp.float32),
            jnp.int32(0))
    cent, _, _ = jax.lax.fori_loop(0, npoint, body, init)
    out_ref[...] = cent.reshape(1, 1, npoint)


def _fps(xyzg, npoint):
    B, _, _, N8 = xyzg.shape
    N = N8 * 8
    return pl.pallas_call(
        functools.partial(_fps_body, npoint=npoint, N=N),
        grid=(B,),
        in_specs=[pl.BlockSpec((1, 3, 8, N8), lambda b: (b, 0, 0, 0))],
        out_specs=pl.BlockSpec((1, 1, npoint), lambda b: (b, 0, 0)),
        out_shape=jax.ShapeDtypeStruct((B, 1, npoint), jnp.int32),
    )(xyzg)


def _ball_body(keys_ref, q_ref, out_ref, *, radius, nsample, N, TQ):
    k3n = keys_ref[0]
    q3 = q_ref[0]
    dot = jax.lax.dot(q3, k3n, preferred_element_type=jnp.float32)
    sq = jnp.sum(q3 * q3, axis=1, keepdims=True)
    sk = jnp.sum(k3n * k3n, axis=0, keepdims=True)
    d2 = sq + sk - 2.0 * dot

    maskf = jnp.where(d2 <= radius * radius, 1.0, 0.0).astype(jnp.float32)

    li = jax.lax.broadcasted_iota(jnp.int32, (128, 128), 0)
    lj = jax.lax.broadcasted_iota(jnp.int32, (128, 128), 1)
    T128 = (li <= lj).astype(jnp.float32)
    NB = N // 128
    pieces = []
    run = jnp.zeros((TQ, 1), jnp.float32)
    for b in range(NB):
        mb = maskf[:, b * 128:(b + 1) * 128]
        incl = jax.lax.dot(mb, T128, precision=jax.lax.Precision.HIGHEST,
                           preferred_element_type=jnp.float32)
        pieces.append(incl + run)
        run = run + incl[:, 127:128]
    cnt = jnp.concatenate(pieces, axis=1)

    Nf = jnp.float32(N)
    j0 = jnp.sum(jnp.where(cnt <= 0.0, 1.0, 0.0), axis=1)
    for k in range(nsample):
        if k == 0:
            jk = j0
        else:
            jk = jnp.sum(jnp.where(cnt <= jnp.float32(k), 1.0, 0.0), axis=1)
        jk = jnp.where(jk == Nf, j0, jk)
        jk = jnp.minimum(jk, Nf - 1.0)
        out_ref[0, k, :] = jk.astype(jnp.int32)


def _ball_select(xyzT, newxyz, radius, nsample):
    B, _, N = xyzT.shape
    S = newxyz.shape[1]
    TQ = min(S, 256)
    return pl.pallas_call(
        functools.partial(_ball_body, radius=radius, nsample=nsample, N=N,
                          TQ=TQ),
        grid=(B, S // TQ),
        in_specs=[
            pl.BlockSpec((1, 3, N), lambda b, t: (b, 0, 0)),
            pl.BlockSpec((1, TQ, 3), lambda b, t: (b, t, 0)),
        ],
        out_specs=pl.BlockSpec((1, nsample, TQ), lambda b, t: (b, 0, t)),
        out_shape=jax.ShapeDtypeStruct((B, nsample, S), jnp.int32),
    )(xyzT, newxyz)


def _mm_body(x_ref, w_ref, a_ref, b_ref, out_ref, *, relu):
    y = jax.lax.dot(x_ref[...], w_ref[...],
                    preferred_element_type=jnp.float32)
    y = y * a_ref[...] + b_ref[...]
    if relu:
        y = jnp.maximum(y, 0.0)
    out_ref[...] = y


def _mm(x, wT, alpha, beta, relu):
    M, C = x.shape
    O = wT.shape[1]
    TM = min(M, 512)
    return pl.pallas_call(
        functools.partial(_mm_body, relu=relu),
        grid=(M // TM,),
        in_specs=[
            pl.BlockSpec((TM, C), lambda i: (i, 0)),
            pl.BlockSpec((C, O), lambda i: (0, 0)),
            pl.BlockSpec((1, O), lambda i: (0, 0)),
            pl.BlockSpec((1, O), lambda i: (0, 0)),
        ],
        out_specs=pl.BlockSpec((TM, O), lambda i: (i, 0)),
        out_shape=jax.ShapeDtypeStruct((M, O), jnp.float32),
    )(x, wT, alpha, beta)


def _gather8_body(feat_ref, sub_ref, idx_ref, out_ref, *, N, TP):
    feat = feat_ref[0]
    sub = sub_ref[0]
    idx = idx_ref[0]
    kiota = jax.lax.broadcasted_iota(jnp.int32, (TP, N), 1)
    for o in range(8):
        col = idx[:, o:o + 1]
        onehot = (kiota == col).astype(jnp.float32)
        g = jax.lax.dot(onehot, feat, precision=jax.lax.Precision.HIGHEST,
                        preferred_element_type=jnp.float32)
        out_ref[0, :, o, :] = g - sub


def _gather8(feat, featsub, idx8):
    B, N, C = feat.shape
    TP = min(N, 128)
    return pl.pallas_call(
        functools.partial(_gather8_body, N=N, TP=TP),
        grid=(B, N // TP),
        in_specs=[
            pl.BlockSpec((1, N, C), lambda b, t: (b, 0, 0)),
            pl.BlockSpec((1, TP, C), lambda b, t: (b, t, 0)),
            pl.BlockSpec((1, TP, 8), lambda b, t: (b, t, 0)),
        ],
        out_specs=pl.BlockSpec((1, TP, 8, C), lambda b, t: (b, t, 0, 0)),
        out_shape=jax.ShapeDtypeStruct((B, N, 8, C), jnp.float32),
    )(feat, featsub, idx8)


def _sagm_body(z_ref, idx_ref, w_ref, a_ref, b_ref, out_ref, *, N, TQ, ns):
    z = z_ref[0]
    idx = idx_ref[0]
    kiota = jax.lax.broadcasted_iota(jnp.int32, (TQ, N), 1)
    acc = None
    for k in range(ns):
        col = idx[:, k:k + 1]
        onehot = (kiota == col).astype(jnp.float32)
        g = jax.lax.dot(onehot, z, precision=jax.lax.Precision.HIGHEST,
                        preferred_element_type=jnp.float32)
        acc = g if acc is None else jnp.maximum(acc, g)
    y = (acc - w_ref[0]) * a_ref[...] + b_ref[...]
    out_ref[0] = jnp.maximum(y, 0.0)


def _sa_gathermax(z, idx, w, alpha, beta):
    B, N, O = z.shape
    S, ns = idx.shape[1], idx.shape[2]
    TQ = min(S, 128)
    return pl.pallas_call(
        functools.partial(_sagm_body, N=N, TQ=TQ, ns=ns),
        grid=(B, S // TQ),
        in_specs=[
            pl.BlockSpec((1, N, O), lambda b, t: (b, 0, 0)),
            pl.BlockSpec((1, TQ, ns), lambda b, t: (b, t, 0)),
            pl.BlockSpec((1, TQ, O), lambda b, t: (b, t, 0)),
            pl.BlockSpec((1, O), lambda b, t: (0, 0)),
            pl.BlockSpec((1, O), lambda b, t: (0, 0)),
        ],
        out_specs=pl.BlockSpec((1, TQ, O), lambda b, t: (b, t, 0)),
        out_shape=jax.ShapeDtypeStruct((B, S, O), jnp.float32),
    )(z, idx, w, alpha, beta)


def _samax_body(x_ref, w_ref, a_ref, b_ref, out_ref):
    y = jax.lax.dot(x_ref[0], w_ref[...], preferred_element_type=jnp.float32)
    y = jnp.maximum(y * a_ref[...] + b_ref[...], 0.0)
    out_ref[...] = jnp.max(y, axis=0).reshape(1, 1, -1)


def _samax_dense(feat, wT, alpha, beta):
    B, N, C = feat.shape
    O = wT.shape[1]
    return pl.pallas_call(
        _samax_body,
        grid=(B,),
        in_specs=[
            pl.BlockSpec((1, N, C), lambda b: (b, 0, 0)),
            pl.BlockSpec((C, O), lambda b: (0, 0)),
            pl.BlockSpec((1, O), lambda b: (0, 0)),
            pl.BlockSpec((1, O), lambda b: (0, 0)),
        ],
        out_specs=pl.BlockSpec((1, 1, O), lambda b: (b, 0, 0)),
        out_shape=jax.ShapeDtypeStruct((B, 1, O), jnp.float32),
    )(feat, wT, alpha, beta)


def _growt_body(tab_ref, idx_ref, out_ref, *, N, S):
    idx = idx_ref[0]
    kiota = jax.lax.broadcasted_iota(jnp.int32, (S, N), 1)
    onehot = (kiota == idx).astype(jnp.float32)
    out_ref[0] = jax.lax.dot(onehot, tab_ref[0],
                             precision=jax.lax.Precision.HIGHEST,
                             preferred_element_type=jnp.float32)


def _gather_rows(tab, idxcol):
    B, N, C = tab.shape
    S = idxcol.shape[1]
    return pl.pallas_call(
        functools.partial(_growt_body, N=N, S=S),
        grid=(B,),
        in_specs=[
            pl.BlockSpec((1, N, C), lambda b: (b, 0, 0)),
            pl.BlockSpec((1, S, 1), lambda b: (b, 0, 0)),
        ],
        out_specs=pl.BlockSpec((1, S, C), lambda b: (b, 0, 0)),
        out_shape=jax.ShapeDtypeStruct((B, S, C), jnp.float32),
    )(tab, idxcol)


def _ab(g, b):
    return (g * _INV_SQRT).reshape(1, -1), b.reshape(1, -1)


def _oe_flat(W):
    return jnp.transpose(W, (2, 1, 0)).reshape(-1, W.shape[0])


def _pointsift(radius, xyz, pts, p):
    B, N, _ = xyz.shape
    O = p['W1'].shape[0]
    xyzT = jnp.transpose(xyz, (0, 2, 1))
    idx8 = jnp.transpose(_sift_select(xyzT, xyz, radius), (0, 2, 1))

    if pts is None:
        feat = xyz
        featsub = xyz
    else:
        feat = jnp.concatenate([xyz, pts], axis=-1)
        featsub = jnp.concatenate([xyz, jnp.zeros_like(pts)], axis=-1)
    C = feat.shape[-1]

    grouped = _gather8(feat, featsub, idx8)
    a1, b1 = _ab(p['g1'], p['b1'])
    y1 = _mm(grouped.reshape(B * N * 4, 2 * C), _oe_flat(p['W1']),
             a1, b1, True)
    a2, b2 = _ab(p['g2'], p['b2'])
    y2 = _mm(y1.reshape(B * N * 2, 2 * O), _oe_flat(p['W2']), a2, b2, True)
    a3, b3 = _ab(p['g3'], p['b3'])
    y3 = _mm(y2.reshape(B * N, 2 * O), _oe_flat(p['W3']), a3, b3, True)
    return y3.reshape(B, N, O)


def _pointnet_sa(npoint, radius, nsample, xyz, pts, p):
    B, N, _ = xyz.shape
    O, C = p['W'].shape
    xyzT = jnp.transpose(xyz, (0, 2, 1))
    fps_idx = _fps(xyzT.reshape(B, 3, 8, N // 8), npoint)
    new_xyz = _gather_rows(xyz, fps_idx.reshape(B, npoint, 1))
    ballT = _ball_select(xyzT, new_xyz, radius, nsample)
    ball = jnp.transpose(ballT, (0, 2, 1))

    feat = jnp.concatenate([xyz, pts], axis=-1)
    ones = jnp.ones((1, O), jnp.float32)
    zeros = jnp.zeros((1, O), jnp.float32)
    z = _mm(feat.reshape(B * N, C), p['W'].T, ones, zeros,
            False).reshape(B, N, O)
    w = _mm(new_xyz.reshape(B * npoint, 3), p['W'][:, :3].T, ones, zeros,
            False).reshape(B, npoint, O)
    a, b = _ab(p['g'], p['b'])
    out = _sa_gathermax(z, ball, w, a, b)
    return new_xyz, out


def kernel(xyz, params):
    B = xyz.shape[0]
    pts = _pointsift(4.0, xyz, None, params['ps1'])
    xyz, pts = _pointnet_sa(1024, 4.0, 32, xyz, pts, params['sa1'])
    pts = _pointsift(6.0, xyz, pts, params['ps2'])
    xyz, pts = _pointnet_sa(256, 6.0, 16, xyz, pts, params['sa2'])
    pts = _pointsift(8.0, xyz, pts, params['ps3'])
    xyz, pts = _pointnet_sa(64, 8.0, 8, xyz, pts, params['sa3'])
    pts = _pointsift(10.0, xyz, pts, params['ps4'])

    feat = jnp.concatenate([xyz, pts], axis=-1)
    p4 = params['sa4']
    a4, b4 = _ab(p4['g'], p4['b'])
    out = _samax_dense(feat, p4['W'].T, a4, b4)
    return out.reshape(B, -1)


# speedup vs baseline: 1.7245x; 1.7245x over previous
import functools
import math

import jax
import jax.numpy as jnp
from jax import lax
from jax.experimental import pallas as pl
from jax.experimental.pallas import tpu as pltpu
import jax.experimental.pallas.tpu_sc as plsc

EPS = 1e-5
_INV_SQRT = 1.0 / math.sqrt(1.0 + EPS)


def _sift_select_body(keys_ref, q_ref, out_ref, *, radius, N, TQ):
    k3n = keys_ref[0]
    q3 = q_ref[0]
    dot = jax.lax.dot(q3, k3n, preferred_element_type=jnp.float32)
    sq = jnp.sum(q3 * q3, axis=1, keepdims=True)
    sk = jnp.sum(k3n * k3n, axis=0, keepdims=True)
    d2 = sq + sk - 2.0 * dot

    gx = (k3n[0:1, :] > q3[:, 0:1]).astype(jnp.int32)
    gy = (k3n[1:2, :] > q3[:, 1:2]).astype(jnp.int32)
    gz = (k3n[2:3, :] > q3[:, 2:3]).astype(jnp.int32)
    octant = gx * 4 + gy * 2 + gz

    qbase = pl.program_id(1) * TQ
    col = jax.lax.broadcasted_iota(jnp.int32, (TQ, N), 1)
    rowg = jax.lax.broadcasted_iota(jnp.int32, (TQ, N), 0) + qbase
    valid = (d2 <= radius * radius) & (col != rowg)
    self_idx = (jax.lax.broadcasted_iota(jnp.int32, (TQ, 1), 0) + qbase)[:, 0]

    colf = col.astype(jnp.float32)
    INF = jnp.float32(jnp.inf)
    for o in range(8):
        d_o = jnp.where(valid & (octant == o), d2, INF)
        m_o = jnp.min(d_o, axis=1, keepdims=True)
        j_o = jnp.min(jnp.where(d_o == m_o, colf, jnp.float32(N)),
                      axis=1)
        found = m_o[:, 0] < INF
        out_ref[0, o, :] = jnp.where(found, j_o.astype(jnp.int32), self_idx)


def _sift_select(xyzT, xyz, radius):
    B, _, N = xyzT.shape
    TQ = min(N, 256)
    grid = (B, N // TQ)
    return pl.pallas_call(
        functools.partial(_sift_select_body, radius=radius, N=N, TQ=TQ),
        grid=grid,
        in_specs=[
            pl.BlockSpec((1, 3, N), lambda b, t: (b, 0, 0)),
            pl.BlockSpec((1, TQ, 3), lambda b, t: (b, t, 0)),
        ],
        out_specs=pl.BlockSpec((1, 8, TQ), lambda b, t: (b, 0, t)),
        out_shape=jax.ShapeDtypeStruct((B, 8, N), jnp.int32),
    )(xyzT, xyz)


def _fps_body(xyzg_ref, out_ref, *, npoint, N):
    x = xyzg_ref[0, 0]
    y = xyzg_ref[0, 1]
    z = xyzg_ref[0, 2]
    N8 = N // 8
    giota = (jax.lax.broadcasted_iota(jnp.int32, (8, N8), 0) * N8
             + jax.lax.broadcasted_iota(jnp.int32, (8, N8), 1))
    np_iota = jax.lax.broadcasted_iota(jnp.int32, (1, npoint), 1)

    def body(i, state):
        cent, dist, far = state
        cent = jnp.where(np_iota == i, far, cent)
        m = giota == far
        cx = jnp.sum(jnp.where(m, x, 0.0))
        cy = jnp.sum(jnp.where(m, y, 0.0))
        cz = jnp.sum(jnp.where(m, z, 0.0))
        dx = x - cx
        dy = y - cy
        dz = z - cz
        dn = dx * dx + dy * dy + dz * dz
        dist = jnp.minimum(dist, dn)
        mx = jnp.max(dist)
        far_new = jnp.min(jnp.where(dist == mx, giota, jnp.int32(N)))
        return cent, dist, far_new

    init = (jnp.zeros((1, npoint), jnp.int32),
            jnp.full((8, N8), 1e10, jnp.float32),
            jnp.int32(0))
    cent, _, _ = jax.lax.fori_loop(0, npoint, body, init)
    out_ref[...] = cent.reshape(1, 1, npoint)


def _fps(xyzg, npoint):
    B, _, _, N8 = xyzg.shape
    N = N8 * 8
    return pl.pallas_call(
        functools.partial(_fps_body, npoint=npoint, N=N),
        grid=(B,),
        in_specs=[pl.BlockSpec((1, 3, 8, N8), lambda b: (b, 0, 0, 0))],
        out_specs=pl.BlockSpec((1, 1, npoint), lambda b: (b, 0, 0)),
        out_shape=jax.ShapeDtypeStruct((B, 1, npoint), jnp.int32),
    )(xyzg)


def _ball_body(keys_ref, q_ref, out_ref, *, radius, nsample, N, TQ):
    k3n = keys_ref[0]
    q3 = q_ref[0]
    dot = jax.lax.dot(q3, k3n, preferred_element_type=jnp.float32)
    sq = jnp.sum(q3 * q3, axis=1, keepdims=True)
    sk = jnp.sum(k3n * k3n, axis=0, keepdims=True)
    d2 = sq + sk - 2.0 * dot

    maskf = jnp.where(d2 <= radius * radius, 1.0, 0.0).astype(jnp.float32)

    li = jax.lax.broadcasted_iota(jnp.int32, (128, 128), 0)
    lj = jax.lax.broadcasted_iota(jnp.int32, (128, 128), 1)
    T128 = (li <= lj).astype(jnp.float32)
    NB = N // 128
    pieces = []
    run = jnp.zeros((TQ, 1), jnp.float32)
    for b in range(NB):
        mb = maskf[:, b * 128:(b + 1) * 128]
        incl = jax.lax.dot(mb, T128, preferred_element_type=jnp.float32)
        pieces.append(incl + run)
        run = run + incl[:, 127:128]
    cnt = jnp.concatenate(pieces, axis=1)

    Nf = jnp.float32(N)
    j0 = jnp.sum(jnp.where(cnt <= 0.0, 1.0, 0.0), axis=1)
    for k in range(nsample):
        if k == 0:
            jk = j0
        else:
            jk = jnp.sum(jnp.where(cnt <= jnp.float32(k), 1.0, 0.0), axis=1)
        jk = jnp.where(jk == Nf, j0, jk)
        jk = jnp.minimum(jk, Nf - 1.0)
        out_ref[0, k, :] = jk.astype(jnp.int32)


def _ball_select(xyzT, newxyz, radius, nsample):
    B, _, N = xyzT.shape
    S = newxyz.shape[1]
    TQ = min(S, 256)
    return pl.pallas_call(
        functools.partial(_ball_body, radius=radius, nsample=nsample, N=N,
                          TQ=TQ),
        grid=(B, S // TQ),
        in_specs=[
            pl.BlockSpec((1, 3, N), lambda b, t: (b, 0, 0)),
            pl.BlockSpec((1, TQ, 3), lambda b, t: (b, t, 0)),
        ],
        out_specs=pl.BlockSpec((1, nsample, TQ), lambda b, t: (b, 0, t)),
        out_shape=jax.ShapeDtypeStruct((B, nsample, S), jnp.int32),
    )(xyzT, newxyz)


def _mm_body(x_ref, w_ref, a_ref, b_ref, out_ref, *, relu):
    y = jax.lax.dot(x_ref[...], w_ref[...],
                    preferred_element_type=jnp.float32)
    y = y * a_ref[...] + b_ref[...]
    if relu:
        y = jnp.maximum(y, 0.0)
    out_ref[...] = y


def _mm(x, wT, alpha, beta, relu):
    M, C = x.shape
    O = wT.shape[1]
    TM = min(M, 512)
    return pl.pallas_call(
        functools.partial(_mm_body, relu=relu),
        grid=(M // TM,),
        in_specs=[
            pl.BlockSpec((TM, C), lambda i: (i, 0)),
            pl.BlockSpec((C, O), lambda i: (0, 0)),
            pl.BlockSpec((1, O), lambda i: (0, 0)),
            pl.BlockSpec((1, O), lambda i: (0, 0)),
        ],
        out_specs=pl.BlockSpec((TM, O), lambda i: (i, 0)),
        out_shape=jax.ShapeDtypeStruct((M, O), jnp.float32),
    )(x, wT, alpha, beta)


def _gather8_body(feat_ref, sub_ref, idx_ref, out_ref, *, N, TP):
    feat = feat_ref[0]
    sub = sub_ref[0]
    idx = idx_ref[0]
    kiota = jax.lax.broadcasted_iota(jnp.int32, (TP, N), 1)
    for o in range(8):
        col = idx[:, o:o + 1]
        onehot = (kiota == col).astype(jnp.float32)
        g = jax.lax.dot(onehot, feat, preferred_element_type=jnp.float32)
        out_ref[0, :, o, :] = g - sub


def _gather8(feat, featsub, idx8):
    B, N, C = feat.shape
    TP = min(N, 128)
    return pl.pallas_call(
        functools.partial(_gather8_body, N=N, TP=TP),
        grid=(B, N // TP),
        in_specs=[
            pl.BlockSpec((1, N, C), lambda b, t: (b, 0, 0)),
            pl.BlockSpec((1, TP, C), lambda b, t: (b, t, 0)),
            pl.BlockSpec((1, TP, 8), lambda b, t: (b, t, 0)),
        ],
        out_specs=pl.BlockSpec((1, TP, 8, C), lambda b, t: (b, t, 0, 0)),
        out_shape=jax.ShapeDtypeStruct((B, N, 8, C), jnp.float32),
    )(feat, featsub, idx8)


def _sagm_body(z_ref, idx_ref, w_ref, a_ref, b_ref, out_ref, *, N, TQ, ns):
    z = z_ref[0]
    idx = idx_ref[0]
    kiota = jax.lax.broadcasted_iota(jnp.int32, (TQ, N), 1)
    acc = None
    for k in range(ns):
        col = idx[:, k:k + 1]
        onehot = (kiota == col).astype(jnp.float32)
        g = jax.lax.dot(onehot, z, preferred_element_type=jnp.float32)
        acc = g if acc is None else jnp.maximum(acc, g)
    y = (acc - w_ref[0]) * a_ref[...] + b_ref[...]
    out_ref[0] = jnp.maximum(y, 0.0)


def _sa_gathermax(z, idx, w, alpha, beta):
    B, N, O = z.shape
    S, ns = idx.shape[1], idx.shape[2]
    TQ = min(S, 128)
    return pl.pallas_call(
        functools.partial(_sagm_body, N=N, TQ=TQ, ns=ns),
        grid=(B, S // TQ),
        in_specs=[
            pl.BlockSpec((1, N, O), lambda b, t: (b, 0, 0)),
            pl.BlockSpec((1, TQ, ns), lambda b, t: (b, t, 0)),
            pl.BlockSpec((1, TQ, O), lambda b, t: (b, t, 0)),
            pl.BlockSpec((1, O), lambda b, t: (0, 0)),
            pl.BlockSpec((1, O), lambda b, t: (0, 0)),
        ],
        out_specs=pl.BlockSpec((1, TQ, O), lambda b, t: (b, t, 0)),
        out_shape=jax.ShapeDtypeStruct((B, S, O), jnp.float32),
    )(z, idx, w, alpha, beta)


def _samax_body(x_ref, w_ref, a_ref, b_ref, out_ref):
    y = jax.lax.dot(x_ref[0], w_ref[...], preferred_element_type=jnp.float32)
    y = jnp.maximum(y * a_ref[...] + b_ref[...], 0.0)
    out_ref[...] = jnp.max(y, axis=0).reshape(1, 1, -1)


def _samax_dense(feat, wT, alpha, beta):
    B, N, C = feat.shape
    O = wT.shape[1]
    return pl.pallas_call(
        _samax_body,
        grid=(B,),
        in_specs=[
            pl.BlockSpec((1, N, C), lambda b: (b, 0, 0)),
            pl.BlockSpec((C, O), lambda b: (0, 0)),
            pl.BlockSpec((1, O), lambda b: (0, 0)),
            pl.BlockSpec((1, O), lambda b: (0, 0)),
        ],
        out_specs=pl.BlockSpec((1, 1, O), lambda b: (b, 0, 0)),
        out_shape=jax.ShapeDtypeStruct((B, 1, O), jnp.float32),
    )(feat, wT, alpha, beta)


def _growt_body(tab_ref, idx_ref, out_ref, *, N, S):
    idx = idx_ref[0]
    kiota = jax.lax.broadcasted_iota(jnp.int32, (S, N), 1)
    onehot = (kiota == idx).astype(jnp.float32)
    out_ref[0] = jax.lax.dot(onehot, tab_ref[0],
                             precision=jax.lax.Precision.HIGHEST,
                             preferred_element_type=jnp.float32)


def _gather_rows(tab, idxcol):
    B, N, C = tab.shape
    S = idxcol.shape[1]
    return pl.pallas_call(
        functools.partial(_growt_body, N=N, S=S),
        grid=(B,),
        in_specs=[
            pl.BlockSpec((1, N, C), lambda b: (b, 0, 0)),
            pl.BlockSpec((1, S, 1), lambda b: (b, 0, 0)),
        ],
        out_specs=pl.BlockSpec((1, S, C), lambda b: (b, 0, 0)),
        out_shape=jax.ShapeDtypeStruct((B, S, C), jnp.float32),
    )(tab, idxcol)


def _ab(g, b):
    return (g * _INV_SQRT).reshape(1, -1), b.reshape(1, -1)


def _oe_flat(W):
    return jnp.transpose(W, (2, 1, 0)).reshape(-1, W.shape[0])


def _pointsift(radius, xyz, pts, p):
    B, N, _ = xyz.shape
    O = p['W1'].shape[0]
    xyzT = jnp.transpose(xyz, (0, 2, 1))
    idx8 = jnp.transpose(_sift_select(xyzT, xyz, radius), (0, 2, 1))

    if pts is None:
        feat = xyz
        featsub = xyz
    else:
        feat = jnp.concatenate([xyz, pts], axis=-1)
        featsub = jnp.concatenate([xyz, jnp.zeros_like(pts)], axis=-1)
    C = feat.shape[-1]

    grouped = _gather8(feat, featsub, idx8)
    a1, b1 = _ab(p['g1'], p['b1'])
    y1 = _mm(grouped.reshape(B * N * 4, 2 * C), _oe_flat(p['W1']),
             a1, b1, True)
    a2, b2 = _ab(p['g2'], p['b2'])
    y2 = _mm(y1.reshape(B * N * 2, 2 * O), _oe_flat(p['W2']), a2, b2, True)
    a3, b3 = _ab(p['g3'], p['b3'])
    y3 = _mm(y2.reshape(B * N, 2 * O), _oe_flat(p['W3']), a3, b3, True)
    return y3.reshape(B, N, O)


def _pointnet_sa(npoint, radius, nsample, xyz, pts, p):
    B, N, _ = xyz.shape
    O, C = p['W'].shape
    xyzT = jnp.transpose(xyz, (0, 2, 1))
    fps_idx = _fps(xyzT.reshape(B, 3, 8, N // 8), npoint)
    new_xyz = _gather_rows(xyz, fps_idx.reshape(B, npoint, 1))
    ballT = _ball_select(xyzT, new_xyz, radius, nsample)
    ball = jnp.transpose(ballT, (0, 2, 1))

    feat = jnp.concatenate([xyz, pts], axis=-1)
    ones = jnp.ones((1, O), jnp.float32)
    zeros = jnp.zeros((1, O), jnp.float32)
    z = _mm(feat.reshape(B * N, C), p['W'].T, ones, zeros,
            False).reshape(B, N, O)
    w = _mm(new_xyz.reshape(B * npoint, 3), p['W'][:, :3].T, ones, zeros,
            False).reshape(B, npoint, O)
    a, b = _ab(p['g'], p['b'])
    out = _sa_gathermax(z, ball, w, a, b)
    return new_xyz, out


def kernel(xyz, params):
    B = xyz.shape[0]
    pts = _pointsift(4.0, xyz, None, params['ps1'])
    xyz, pts = _pointnet_sa(1024, 4.0, 32, xyz, pts, params['sa1'])
    pts = _pointsift(6.0, xyz, pts, params['ps2'])
    xyz, pts = _pointnet_sa(256, 6.0, 16, xyz, pts, params['sa2'])
    pts = _pointsift(8.0, xyz, pts, params['ps3'])
    xyz, pts = _pointnet_sa(64, 8.0, 8, xyz, pts, params['sa3'])
    pts = _pointsift(10.0, xyz, pts, params['ps4'])

    feat = jnp.concatenate([xyz, pts], axis=-1)
    p4 = params['sa4']
    a4, b4 = _ab(p4['g'], p4['b'])
    out = _samax_dense(feat, p4['W'].T, a4, b4)
    return out.reshape(B, -1)


# speedup vs baseline: 1.7652x; 1.0236x over previous
import functools
import math

import jax
import jax.numpy as jnp
from jax import lax
from jax.experimental import pallas as pl
from jax.experimental.pallas import tpu as pltpu
import jax.experimental.pallas.tpu_sc as plsc

EPS = 1e-5
_INV_SQRT = 1.0 / math.sqrt(1.0 + EPS)


def _sift_select_body(keys_ref, q_ref, out_ref, *, radius, N, TQ):
    k3n = keys_ref[0]
    q3 = q_ref[0]
    dot = jax.lax.dot(q3, k3n, preferred_element_type=jnp.float32)
    sq = jnp.sum(q3 * q3, axis=1, keepdims=True)
    sk = jnp.sum(k3n * k3n, axis=0, keepdims=True)
    d2 = sq + sk - 2.0 * dot

    gx = (k3n[0:1, :] > q3[:, 0:1]).astype(jnp.int32)
    gy = (k3n[1:2, :] > q3[:, 1:2]).astype(jnp.int32)
    gz = (k3n[2:3, :] > q3[:, 2:3]).astype(jnp.int32)
    octant = gx * 4 + gy * 2 + gz

    qbase = pl.program_id(1) * TQ
    col = jax.lax.broadcasted_iota(jnp.int32, (TQ, N), 1)
    rowg = jax.lax.broadcasted_iota(jnp.int32, (TQ, N), 0) + qbase
    valid = (d2 <= radius * radius) & (col != rowg)
    self_idx = (jax.lax.broadcasted_iota(jnp.int32, (TQ, 1), 0) + qbase)[:, 0]

    colf = col.astype(jnp.float32)
    INF = jnp.float32(jnp.inf)
    for o in range(8):
        d_o = jnp.where(valid & (octant == o), d2, INF)
        m_o = jnp.min(d_o, axis=1, keepdims=True)
        j_o = jnp.min(jnp.where(d_o == m_o, colf, jnp.float32(N)),
                      axis=1)
        found = m_o[:, 0] < INF
        out_ref[0, o, :] = jnp.where(found, j_o.astype(jnp.int32), self_idx)


def _sift_select(xyzT, xyz, radius):
    B, _, N = xyzT.shape
    TQ = min(N, 256)
    grid = (B, N // TQ)
    return pl.pallas_call(
        functools.partial(_sift_select_body, radius=radius, N=N, TQ=TQ),
        grid=grid,
        in_specs=[
            pl.BlockSpec((1, 3, N), lambda b, t: (b, 0, 0)),
            pl.BlockSpec((1, TQ, 3), lambda b, t: (b, t, 0)),
        ],
        out_specs=pl.BlockSpec((1, 8, TQ), lambda b, t: (b, 0, t)),
        out_shape=jax.ShapeDtypeStruct((B, 8, N), jnp.int32),
    )(xyzT, xyz)


def _fps_body(xyzg_ref, rows_ref, out_ref, *, npoint, N):
    x = xyzg_ref[0, 0]
    y = xyzg_ref[0, 1]
    z = xyzg_ref[0, 2]
    N8 = N // 8
    giota = (jax.lax.broadcasted_iota(jnp.int32, (8, N8), 0) * N8
             + jax.lax.broadcasted_iota(jnp.int32, (8, N8), 1))
    np_iota = jax.lax.broadcasted_iota(jnp.int32, (1, npoint), 1)

    def body(i, state):
        cent, dist, far = state
        cent = jnp.where(np_iota == i, far, cent)
        row = rows_ref[0, pl.ds(far, 1), :]
        dx = x - row[0, 0]
        dy = y - row[0, 1]
        dz = z - row[0, 2]
        dn = dx * dx + dy * dy + dz * dz
        dist = jnp.minimum(dist, dn)
        mx = jnp.max(dist)
        far_new = jnp.min(jnp.where(dist == mx, giota, jnp.int32(N)))
        return cent, dist, far_new

    init = (jnp.zeros((1, npoint), jnp.int32),
            jnp.full((8, N8), 1e10, jnp.float32),
            jnp.int32(0))
    cent, _, _ = jax.lax.fori_loop(0, npoint, body, init)
    out_ref[...] = cent.reshape(1, 1, npoint)


def _fps(xyzg, xyz, npoint):
    B, _, _, N8 = xyzg.shape
    N = N8 * 8
    return pl.pallas_call(
        functools.partial(_fps_body, npoint=npoint, N=N),
        grid=(B,),
        in_specs=[
            pl.BlockSpec((1, 3, 8, N8), lambda b: (b, 0, 0, 0)),
            pl.BlockSpec((1, N, 3), lambda b: (b, 0, 0)),
        ],
        out_specs=pl.BlockSpec((1, 1, npoint), lambda b: (b, 0, 0)),
        out_shape=jax.ShapeDtypeStruct((B, 1, npoint), jnp.int32),
    )(xyzg, xyz)


def _ball_body(keys_ref, q_ref, out_ref, *, radius, nsample, N, TQ):
    k3n = keys_ref[0]
    q3 = q_ref[0]
    dot = jax.lax.dot(q3, k3n, preferred_element_type=jnp.float32)
    sq = jnp.sum(q3 * q3, axis=1, keepdims=True)
    sk = jnp.sum(k3n * k3n, axis=0, keepdims=True)
    d2 = sq + sk - 2.0 * dot

    maskf = jnp.where(d2 <= radius * radius, 1.0, 0.0).astype(jnp.float32)

    li = jax.lax.broadcasted_iota(jnp.int32, (128, 128), 0)
    lj = jax.lax.broadcasted_iota(jnp.int32, (128, 128), 1)
    T128 = (li <= lj).astype(jnp.float32)
    NB = N // 128
    pieces = []
    run = jnp.zeros((TQ, 1), jnp.float32)
    for b in range(NB):
        mb = maskf[:, b * 128:(b + 1) * 128]
        incl = jax.lax.dot(mb, T128, preferred_element_type=jnp.float32)
        pieces.append(incl + run)
        run = run + incl[:, 127:128]
    cnt = jnp.concatenate(pieces, axis=1)

    Nf = jnp.float32(N)
    j0 = jnp.sum(jnp.where(cnt <= 0.0, 1.0, 0.0), axis=1)
    for k in range(nsample):
        if k == 0:
            jk = j0
        else:
            jk = jnp.sum(jnp.where(cnt <= jnp.float32(k), 1.0, 0.0), axis=1)
        jk = jnp.where(jk == Nf, j0, jk)
        jk = jnp.minimum(jk, Nf - 1.0)
        out_ref[0, k, :] = jk.astype(jnp.int32)


def _ball_select(xyzT, newxyz, radius, nsample):
    B, _, N = xyzT.shape
    S = newxyz.shape[1]
    TQ = min(S, 256)
    return pl.pallas_call(
        functools.partial(_ball_body, radius=radius, nsample=nsample, N=N,
                          TQ=TQ),
        grid=(B, S // TQ),
        in_specs=[
            pl.BlockSpec((1, 3, N), lambda b, t: (b, 0, 0)),
            pl.BlockSpec((1, TQ, 3), lambda b, t: (b, t, 0)),
        ],
        out_specs=pl.BlockSpec((1, nsample, TQ), lambda b, t: (b, 0, t)),
        out_shape=jax.ShapeDtypeStruct((B, nsample, S), jnp.int32),
    )(xyzT, newxyz)


def _mm_body(x_ref, w_ref, a_ref, b_ref, out_ref, *, relu):
    y = jax.lax.dot(x_ref[...], w_ref[...],
                    preferred_element_type=jnp.float32)
    y = y * a_ref[...] + b_ref[...]
    if relu:
        y = jnp.maximum(y, 0.0)
    out_ref[...] = y


def _mm(x, wT, alpha, beta, relu):
    M, C = x.shape
    O = wT.shape[1]
    TM = min(M, 512)
    return pl.pallas_call(
        functools.partial(_mm_body, relu=relu),
        grid=(M // TM,),
        in_specs=[
            pl.BlockSpec((TM, C), lambda i: (i, 0)),
            pl.BlockSpec((C, O), lambda i: (0, 0)),
            pl.BlockSpec((1, O), lambda i: (0, 0)),
            pl.BlockSpec((1, O), lambda i: (0, 0)),
        ],
        out_specs=pl.BlockSpec((TM, O), lambda i: (i, 0)),
        out_shape=jax.ShapeDtypeStruct((M, O), jnp.float32),
    )(x, wT, alpha, beta)


def _gather8_body(feat_ref, sub_ref, idx_ref, out_ref, *, N, TP):
    feat = feat_ref[0]
    sub = sub_ref[0]
    idx = idx_ref[0]
    kiota = jax.lax.broadcasted_iota(jnp.int32, (TP, N), 1)
    for o in range(8):
        col = idx[:, o:o + 1]
        onehot = (kiota == col).astype(jnp.float32)
        g = jax.lax.dot(onehot, feat, preferred_element_type=jnp.float32)
        out_ref[0, :, o, :] = g - sub


def _gather8(feat, featsub, idx8):
    B, N, C = feat.shape
    TP = min(N, 128)
    return pl.pallas_call(
        functools.partial(_gather8_body, N=N, TP=TP),
        grid=(B, N // TP),
        in_specs=[
            pl.BlockSpec((1, N, C), lambda b, t: (b, 0, 0)),
            pl.BlockSpec((1, TP, C), lambda b, t: (b, t, 0)),
            pl.BlockSpec((1, TP, 8), lambda b, t: (b, t, 0)),
        ],
        out_specs=pl.BlockSpec((1, TP, 8, C), lambda b, t: (b, t, 0, 0)),
        out_shape=jax.ShapeDtypeStruct((B, N, 8, C), jnp.float32),
    )(feat, featsub, idx8)


def _sagm_body(z_ref, idx_ref, w_ref, a_ref, b_ref, out_ref, *, N, TQ, ns):
    z = z_ref[0]
    idx = idx_ref[0]
    kiota = jax.lax.broadcasted_iota(jnp.int32, (TQ, N), 1)
    acc = None
    for k in range(ns):
        col = idx[:, k:k + 1]
        onehot = (kiota == col).astype(jnp.float32)
        g = jax.lax.dot(onehot, z, preferred_element_type=jnp.float32)
        acc = g if acc is None else jnp.maximum(acc, g)
    y = (acc - w_ref[0]) * a_ref[...] + b_ref[...]
    out_ref[0] = jnp.maximum(y, 0.0)


def _sa_gathermax(z, idx, w, alpha, beta):
    B, N, O = z.shape
    S, ns = idx.shape[1], idx.shape[2]
    TQ = min(S, 128)
    return pl.pallas_call(
        functools.partial(_sagm_body, N=N, TQ=TQ, ns=ns),
        grid=(B, S // TQ),
        in_specs=[
            pl.BlockSpec((1, N, O), lambda b, t: (b, 0, 0)),
            pl.BlockSpec((1, TQ, ns), lambda b, t: (b, t, 0)),
            pl.BlockSpec((1, TQ, O), lambda b, t: (b, t, 0)),
            pl.BlockSpec((1, O), lambda b, t: (0, 0)),
            pl.BlockSpec((1, O), lambda b, t: (0, 0)),
        ],
        out_specs=pl.BlockSpec((1, TQ, O), lambda b, t: (b, t, 0)),
        out_shape=jax.ShapeDtypeStruct((B, S, O), jnp.float32),
    )(z, idx, w, alpha, beta)


def _samax_body(x_ref, w_ref, a_ref, b_ref, out_ref):
    y = jax.lax.dot(x_ref[0], w_ref[...], preferred_element_type=jnp.float32)
    y = jnp.maximum(y * a_ref[...] + b_ref[...], 0.0)
    out_ref[...] = jnp.max(y, axis=0).reshape(1, 1, -1)


def _samax_dense(feat, wT, alpha, beta):
    B, N, C = feat.shape
    O = wT.shape[1]
    return pl.pallas_call(
        _samax_body,
        grid=(B,),
        in_specs=[
            pl.BlockSpec((1, N, C), lambda b: (b, 0, 0)),
            pl.BlockSpec((C, O), lambda b: (0, 0)),
            pl.BlockSpec((1, O), lambda b: (0, 0)),
            pl.BlockSpec((1, O), lambda b: (0, 0)),
        ],
        out_specs=pl.BlockSpec((1, 1, O), lambda b: (b, 0, 0)),
        out_shape=jax.ShapeDtypeStruct((B, 1, O), jnp.float32),
    )(feat, wT, alpha, beta)


def _growt_body(tab_ref, idx_ref, out_ref, *, N, S):
    idx = idx_ref[0]
    kiota = jax.lax.broadcasted_iota(jnp.int32, (S, N), 1)
    onehot = (kiota == idx).astype(jnp.float32)
    out_ref[0] = jax.lax.dot(onehot, tab_ref[0],
                             precision=jax.lax.Precision.HIGHEST,
                             preferred_element_type=jnp.float32)


def _gather_rows(tab, idxcol):
    B, N, C = tab.shape
    S = idxcol.shape[1]
    return pl.pallas_call(
        functools.partial(_growt_body, N=N, S=S),
        grid=(B,),
        in_specs=[
            pl.BlockSpec((1, N, C), lambda b: (b, 0, 0)),
            pl.BlockSpec((1, S, 1), lambda b: (b, 0, 0)),
        ],
        out_specs=pl.BlockSpec((1, S, C), lambda b: (b, 0, 0)),
        out_shape=jax.ShapeDtypeStruct((B, S, C), jnp.float32),
    )(tab, idxcol)


def _ab(g, b):
    return (g * _INV_SQRT).reshape(1, -1), b.reshape(1, -1)


def _oe_flat(W):
    return jnp.transpose(W, (2, 1, 0)).reshape(-1, W.shape[0])


def _pointsift(radius, xyz, pts, p):
    B, N, _ = xyz.shape
    O = p['W1'].shape[0]
    xyzT = jnp.transpose(xyz, (0, 2, 1))
    idx8 = jnp.transpose(_sift_select(xyzT, xyz, radius), (0, 2, 1))

    if pts is None:
        feat = xyz
        featsub = xyz
    else:
        feat = jnp.concatenate([xyz, pts], axis=-1)
        featsub = jnp.concatenate([xyz, jnp.zeros_like(pts)], axis=-1)
    C = feat.shape[-1]

    grouped = _gather8(feat, featsub, idx8)
    a1, b1 = _ab(p['g1'], p['b1'])
    y1 = _mm(grouped.reshape(B * N * 4, 2 * C), _oe_flat(p['W1']),
             a1, b1, True)
    a2, b2 = _ab(p['g2'], p['b2'])
    y2 = _mm(y1.reshape(B * N * 2, 2 * O), _oe_flat(p['W2']), a2, b2, True)
    a3, b3 = _ab(p['g3'], p['b3'])
    y3 = _mm(y2.reshape(B * N, 2 * O), _oe_flat(p['W3']), a3, b3, True)
    return y3.reshape(B, N, O)


def _pointnet_sa(npoint, radius, nsample, xyz, pts, p):
    B, N, _ = xyz.shape
    O, C = p['W'].shape
    xyzT = jnp.transpose(xyz, (0, 2, 1))
    fps_idx = _fps(xyzT.reshape(B, 3, 8, N // 8), xyz, npoint)
    new_xyz = _gather_rows(xyz, fps_idx.reshape(B, npoint, 1))
    ballT = _ball_select(xyzT, new_xyz, radius, nsample)
    ball = jnp.transpose(ballT, (0, 2, 1))

    feat = jnp.concatenate([xyz, pts], axis=-1)
    ones = jnp.ones((1, O), jnp.float32)
    zeros = jnp.zeros((1, O), jnp.float32)
    z = _mm(feat.reshape(B * N, C), p['W'].T, ones, zeros,
            False).reshape(B, N, O)
    w = _mm(new_xyz.reshape(B * npoint, 3), p['W'][:, :3].T, ones, zeros,
            False).reshape(B, npoint, O)
    a, b = _ab(p['g'], p['b'])
    out = _sa_gathermax(z, ball, w, a, b)
    return new_xyz, out


def kernel(xyz, params):
    B = xyz.shape[0]
    pts = _pointsift(4.0, xyz, None, params['ps1'])
    xyz, pts = _pointnet_sa(1024, 4.0, 32, xyz, pts, params['sa1'])
    pts = _pointsift(6.0, xyz, pts, params['ps2'])
    xyz, pts = _pointnet_sa(256, 6.0, 16, xyz, pts, params['sa2'])
    pts = _pointsift(8.0, xyz, pts, params['ps3'])
    xyz, pts = _pointnet_sa(64, 8.0, 8, xyz, pts, params['sa3'])
    pts = _pointsift(10.0, xyz, pts, params['ps4'])

    feat = jnp.concatenate([xyz, pts], axis=-1)
    p4 = params['sa4']
    a4, b4 = _ab(p4['g'], p4['b'])
    out = _samax_dense(feat, p4['W'].T, a4, b4)
    return out.reshape(B, -1)


# speedup vs baseline: 1.8709x; 1.0599x over previous
import functools
import math

import jax
import jax.numpy as jnp
from jax import lax
from jax.experimental import pallas as pl
from jax.experimental.pallas import tpu as pltpu
import jax.experimental.pallas.tpu_sc as plsc

EPS = 1e-5
_INV_SQRT = 1.0 / math.sqrt(1.0 + EPS)


def _sift_select_body(keys_ref, q_ref, out_ref, *, radius, N, TQ):
    k3n = keys_ref[0]
    q3 = q_ref[0]
    dot = jax.lax.dot(q3, k3n, preferred_element_type=jnp.float32)
    sq = jnp.sum(q3 * q3, axis=1, keepdims=True)
    sk = jnp.sum(k3n * k3n, axis=0, keepdims=True)
    d2 = sq + sk - 2.0 * dot

    gx = (k3n[0:1, :] > q3[:, 0:1]).astype(jnp.int32)
    gy = (k3n[1:2, :] > q3[:, 1:2]).astype(jnp.int32)
    gz = (k3n[2:3, :] > q3[:, 2:3]).astype(jnp.int32)
    octant = gx * 4 + gy * 2 + gz

    qbase = pl.program_id(1) * TQ
    col = jax.lax.broadcasted_iota(jnp.int32, (TQ, N), 1)
    rowg = jax.lax.broadcasted_iota(jnp.int32, (TQ, N), 0) + qbase
    valid = (d2 <= radius * radius) & (col != rowg)
    self_idx = (jax.lax.broadcasted_iota(jnp.int32, (TQ, 1), 0) + qbase)[:, 0]

    colf = col.astype(jnp.float32)
    INF = jnp.float32(jnp.inf)
    for o in range(8):
        d_o = jnp.where(valid & (octant == o), d2, INF)
        m_o = jnp.min(d_o, axis=1, keepdims=True)
        j_o = jnp.min(jnp.where(d_o == m_o, colf, jnp.float32(N)),
                      axis=1)
        found = m_o[:, 0] < INF
        out_ref[0, o, :] = jnp.where(found, j_o.astype(jnp.int32), self_idx)


def _sift_select(xyzT, xyz, radius):
    B, _, N = xyzT.shape
    TQ = min(N, 256)
    grid = (B, N // TQ)
    return pl.pallas_call(
        functools.partial(_sift_select_body, radius=radius, N=N, TQ=TQ),
        grid=grid,
        in_specs=[
            pl.BlockSpec((1, 3, N), lambda b, t: (b, 0, 0)),
            pl.BlockSpec((1, TQ, 3), lambda b, t: (b, t, 0)),
        ],
        out_specs=pl.BlockSpec((1, 8, TQ), lambda b, t: (b, 0, t)),
        out_shape=jax.ShapeDtypeStruct((B, 8, N), jnp.int32),
    )(xyzT, xyz)


def _fps_body(xyzg_ref, rows_ref, out_ref, *, npoint, N, B):
    N8 = N // 8
    giota = (jax.lax.broadcasted_iota(jnp.int32, (8, N8), 0) * N8
             + jax.lax.broadcasted_iota(jnp.int32, (8, N8), 1))
    np_iota = jax.lax.broadcasted_iota(jnp.int32, (1, npoint), 1)
    xs = [xyzg_ref[b, 0] for b in range(B)]
    ys = [xyzg_ref[b, 1] for b in range(B)]
    zs = [xyzg_ref[b, 2] for b in range(B)]

    def body(i, state):
        out = []
        for b in range(B):
            cent, dist, far = state[b]
            cent = jnp.where(np_iota == i, far, cent)
            row = rows_ref[b, pl.ds(far, 1), :]
            dx = xs[b] - row[0, 0]
            dy = ys[b] - row[0, 1]
            dz = zs[b] - row[0, 2]
            dn = dx * dx + dy * dy + dz * dz
            dist = jnp.minimum(dist, dn)
            mx = jnp.max(dist)
            far_new = jnp.min(jnp.where(dist == mx, giota, jnp.int32(N)))
            out.append((cent, dist, far_new))
        return tuple(out)

    init = tuple((jnp.zeros((1, npoint), jnp.int32),
                  jnp.full((8, N8), 1e10, jnp.float32),
                  jnp.int32(0)) for _ in range(B))
    final = jax.lax.fori_loop(0, npoint, body, init)
    for b in range(B):
        out_ref[b] = final[b][0].reshape(1, npoint)


def _fps(xyzg, xyz, npoint):
    B, _, _, N8 = xyzg.shape
    N = N8 * 8
    return pl.pallas_call(
        functools.partial(_fps_body, npoint=npoint, N=N, B=B),
        grid=(1,),
        in_specs=[
            pl.BlockSpec((B, 3, 8, N8), lambda i: (0, 0, 0, 0)),
            pl.BlockSpec((B, N, 3), lambda i: (0, 0, 0)),
        ],
        out_specs=pl.BlockSpec((B, 1, npoint), lambda i: (0, 0, 0)),
        out_shape=jax.ShapeDtypeStruct((B, 1, npoint), jnp.int32),
    )(xyzg, xyz)


def _ball_body(keys_ref, q_ref, out_ref, *, radius, nsample, N, TQ):
    k3n = keys_ref[0]
    q3 = q_ref[0]
    dot = jax.lax.dot(q3, k3n, preferred_element_type=jnp.float32)
    sq = jnp.sum(q3 * q3, axis=1, keepdims=True)
    sk = jnp.sum(k3n * k3n, axis=0, keepdims=True)
    d2 = sq + sk - 2.0 * dot

    maskf = jnp.where(d2 <= radius * radius, 1.0, 0.0).astype(jnp.float32)

    li = jax.lax.broadcasted_iota(jnp.int32, (128, 128), 0)
    lj = jax.lax.broadcasted_iota(jnp.int32, (128, 128), 1)
    T128 = (li <= lj).astype(jnp.float32)
    NB = N // 128
    pieces = []
    run = jnp.zeros((TQ, 1), jnp.float32)
    for b in range(NB):
        mb = maskf[:, b * 128:(b + 1) * 128]
        incl = jax.lax.dot(mb, T128, preferred_element_type=jnp.float32)
        pieces.append(incl + run)
        run = run + incl[:, 127:128]
    cnt = jnp.concatenate(pieces, axis=1)

    Nf = jnp.float32(N)
    j0 = jnp.sum(jnp.where(cnt <= 0.0, 1.0, 0.0), axis=1)
    for k in range(nsample):
        if k == 0:
            jk = j0
        else:
            jk = jnp.sum(jnp.where(cnt <= jnp.float32(k), 1.0, 0.0), axis=1)
        jk = jnp.where(jk == Nf, j0, jk)
        jk = jnp.minimum(jk, Nf - 1.0)
        out_ref[0, k, :] = jk.astype(jnp.int32)


def _ball_select(xyzT, newxyz, radius, nsample):
    B, _, N = xyzT.shape
    S = newxyz.shape[1]
    TQ = min(S, 256)
    return pl.pallas_call(
        functools.partial(_ball_body, radius=radius, nsample=nsample, N=N,
                          TQ=TQ),
        grid=(B, S // TQ),
        in_specs=[
            pl.BlockSpec((1, 3, N), lambda b, t: (b, 0, 0)),
            pl.BlockSpec((1, TQ, 3), lambda b, t: (b, t, 0)),
        ],
        out_specs=pl.BlockSpec((1, nsample, TQ), lambda b, t: (b, 0, t)),
        out_shape=jax.ShapeDtypeStruct((B, nsample, S), jnp.int32),
    )(xyzT, newxyz)


def _mm_body(x_ref, w_ref, a_ref, b_ref, out_ref, *, relu):
    y = jax.lax.dot(x_ref[...], w_ref[...],
                    preferred_element_type=jnp.float32)
    y = y * a_ref[...] + b_ref[...]
    if relu:
        y = jnp.maximum(y, 0.0)
    out_ref[...] = y


def _mm(x, wT, alpha, beta, relu):
    M, C = x.shape
    O = wT.shape[1]
    TM = min(M, 512)
    return pl.pallas_call(
        functools.partial(_mm_body, relu=relu),
        grid=(M // TM,),
        in_specs=[
            pl.BlockSpec((TM, C), lambda i: (i, 0)),
            pl.BlockSpec((C, O), lambda i: (0, 0)),
            pl.BlockSpec((1, O), lambda i: (0, 0)),
            pl.BlockSpec((1, O), lambda i: (0, 0)),
        ],
        out_specs=pl.BlockSpec((TM, O), lambda i: (i, 0)),
        out_shape=jax.ShapeDtypeStruct((M, O), jnp.float32),
    )(x, wT, alpha, beta)


def _gather8_body(feat_ref, sub_ref, idx_ref, out_ref, *, N, TP):
    feat = feat_ref[0]
    sub = sub_ref[0]
    idx = idx_ref[0]
    kiota = jax.lax.broadcasted_iota(jnp.int32, (TP, N), 1)
    for o in range(8):
        col = idx[:, o:o + 1]
        onehot = (kiota == col).astype(jnp.float32)
        g = jax.lax.dot(onehot, feat, preferred_element_type=jnp.float32)
        out_ref[0, :, o, :] = g - sub


def _gather8(feat, featsub, idx8):
    B, N, C = feat.shape
    TP = min(N, 128)
    return pl.pallas_call(
        functools.partial(_gather8_body, N=N, TP=TP),
        grid=(B, N // TP),
        in_specs=[
            pl.BlockSpec((1, N, C), lambda b, t: (b, 0, 0)),
            pl.BlockSpec((1, TP, C), lambda b, t: (b, t, 0)),
            pl.BlockSpec((1, TP, 8), lambda b, t: (b, t, 0)),
        ],
        out_specs=pl.BlockSpec((1, TP, 8, C), lambda b, t: (b, t, 0, 0)),
        out_shape=jax.ShapeDtypeStruct((B, N, 8, C), jnp.float32),
    )(feat, featsub, idx8)


def _sagm_body(z_ref, idx_ref, w_ref, a_ref, b_ref, out_ref, *, N, TQ, ns):
    z = z_ref[0]
    idx = idx_ref[0]
    kiota = jax.lax.broadcasted_iota(jnp.int32, (TQ, N), 1)
    acc = None
    for k in range(ns):
        col = idx[:, k:k + 1]
        onehot = (kiota == col).astype(jnp.float32)
        g = jax.lax.dot(onehot, z, preferred_element_type=jnp.float32)
        acc = g if acc is None else jnp.maximum(acc, g)
    y = (acc - w_ref[0]) * a_ref[...] + b_ref[...]
    out_ref[0] = jnp.maximum(y, 0.0)


def _sa_gathermax(z, idx, w, alpha, beta):
    B, N, O = z.shape
    S, ns = idx.shape[1], idx.shape[2]
    TQ = min(S, 128)
    return pl.pallas_call(
        functools.partial(_sagm_body, N=N, TQ=TQ, ns=ns),
        grid=(B, S // TQ),
        in_specs=[
            pl.BlockSpec((1, N, O), lambda b, t: (b, 0, 0)),
            pl.BlockSpec((1, TQ, ns), lambda b, t: (b, t, 0)),
            pl.BlockSpec((1, TQ, O), lambda b, t: (b, t, 0)),
            pl.BlockSpec((1, O), lambda b, t: (0, 0)),
            pl.BlockSpec((1, O), lambda b, t: (0, 0)),
        ],
        out_specs=pl.BlockSpec((1, TQ, O), lambda b, t: (b, t, 0)),
        out_shape=jax.ShapeDtypeStruct((B, S, O), jnp.float32),
    )(z, idx, w, alpha, beta)


def _samax_body(x_ref, w_ref, a_ref, b_ref, out_ref):
    y = jax.lax.dot(x_ref[0], w_ref[...], preferred_element_type=jnp.float32)
    y = jnp.maximum(y * a_ref[...] + b_ref[...], 0.0)
    out_ref[...] = jnp.max(y, axis=0).reshape(1, 1, -1)


def _samax_dense(feat, wT, alpha, beta):
    B, N, C = feat.shape
    O = wT.shape[1]
    return pl.pallas_call(
        _samax_body,
        grid=(B,),
        in_specs=[
            pl.BlockSpec((1, N, C), lambda b: (b, 0, 0)),
            pl.BlockSpec((C, O), lambda b: (0, 0)),
            pl.BlockSpec((1, O), lambda b: (0, 0)),
            pl.BlockSpec((1, O), lambda b: (0, 0)),
        ],
        out_specs=pl.BlockSpec((1, 1, O), lambda b: (b, 0, 0)),
        out_shape=jax.ShapeDtypeStruct((B, 1, O), jnp.float32),
    )(feat, wT, alpha, beta)


def _growt_body(tab_ref, idx_ref, out_ref, *, N, S):
    idx = idx_ref[0]
    kiota = jax.lax.broadcasted_iota(jnp.int32, (S, N), 1)
    onehot = (kiota == idx).astype(jnp.float32)
    out_ref[0] = jax.lax.dot(onehot, tab_ref[0],
                             precision=jax.lax.Precision.HIGHEST,
                             preferred_element_type=jnp.float32)


def _gather_rows(tab, idxcol):
    B, N, C = tab.shape
    S = idxcol.shape[1]
    return pl.pallas_call(
        functools.partial(_growt_body, N=N, S=S),
        grid=(B,),
        in_specs=[
            pl.BlockSpec((1, N, C), lambda b: (b, 0, 0)),
            pl.BlockSpec((1, S, 1), lambda b: (b, 0, 0)),
        ],
        out_specs=pl.BlockSpec((1, S, C), lambda b: (b, 0, 0)),
        out_shape=jax.ShapeDtypeStruct((B, S, C), jnp.float32),
    )(tab, idxcol)


def _ab(g, b):
    return (g * _INV_SQRT).reshape(1, -1), b.reshape(1, -1)


def _oe_flat(W):
    return jnp.transpose(W, (2, 1, 0)).reshape(-1, W.shape[0])


def _pointsift(radius, xyz, pts, p):
    B, N, _ = xyz.shape
    O = p['W1'].shape[0]
    xyzT = jnp.transpose(xyz, (0, 2, 1))
    idx8 = jnp.transpose(_sift_select(xyzT, xyz, radius), (0, 2, 1))

    if pts is None:
        feat = xyz
        featsub = xyz
    else:
        feat = jnp.concatenate([xyz, pts], axis=-1)
        featsub = jnp.concatenate([xyz, jnp.zeros_like(pts)], axis=-1)
    C = feat.shape[-1]

    grouped = _gather8(feat, featsub, idx8)
    a1, b1 = _ab(p['g1'], p['b1'])
    y1 = _mm(grouped.reshape(B * N * 4, 2 * C), _oe_flat(p['W1']),
             a1, b1, True)
    a2, b2 = _ab(p['g2'], p['b2'])
    y2 = _mm(y1.reshape(B * N * 2, 2 * O), _oe_flat(p['W2']), a2, b2, True)
    a3, b3 = _ab(p['g3'], p['b3'])
    y3 = _mm(y2.reshape(B * N, 2 * O), _oe_flat(p['W3']), a3, b3, True)
    return y3.reshape(B, N, O)


def _pointnet_sa(npoint, radius, nsample, xyz, pts, p):
    B, N, _ = xyz.shape
    O, C = p['W'].shape
    xyzT = jnp.transpose(xyz, (0, 2, 1))
    fps_idx = _fps(xyzT.reshape(B, 3, 8, N // 8), xyz, npoint)
    new_xyz = _gather_rows(xyz, fps_idx.reshape(B, npoint, 1))
    ballT = _ball_select(xyzT, new_xyz, radius, nsample)
    ball = jnp.transpose(ballT, (0, 2, 1))

    feat = jnp.concatenate([xyz, pts], axis=-1)
    ones = jnp.ones((1, O), jnp.float32)
    zeros = jnp.zeros((1, O), jnp.float32)
    z = _mm(feat.reshape(B * N, C), p['W'].T, ones, zeros,
            False).reshape(B, N, O)
    w = _mm(new_xyz.reshape(B * npoint, 3), p['W'][:, :3].T, ones, zeros,
            False).reshape(B, npoint, O)
    a, b = _ab(p['g'], p['b'])
    out = _sa_gathermax(z, ball, w, a, b)
    return new_xyz, out


def kernel(xyz, params):
    B = xyz.shape[0]
    pts = _pointsift(4.0, xyz, None, params['ps1'])
    xyz, pts = _pointnet_sa(1024, 4.0, 32, xyz, pts, params['sa1'])
    pts = _pointsift(6.0, xyz, pts, params['ps2'])
    xyz, pts = _pointnet_sa(256, 6.0, 16, xyz, pts, params['sa2'])
    pts = _pointsift(8.0, xyz, pts, params['ps3'])
    xyz, pts = _pointnet_sa(64, 8.0, 8, xyz, pts, params['sa3'])
    pts = _pointsift(10.0, xyz, pts, params['ps4'])

    feat = jnp.concatenate([xyz, pts], axis=-1)
    p4 = params['sa4']
    a4, b4 = _ab(p4['g'], p4['b'])
    out = _samax_dense(feat, p4['W'].T, a4, b4)
    return out.reshape(B, -1)
